# baseline (speedup 1.0000x reference)
import functools

import jax
import jax.numpy as jnp
from jax import lax
from jax.experimental import pallas as pl
from jax.experimental.pallas import tpu as pltpu
from jax.experimental.pallas import tpu_sc as plsc

NH = 5
NC = 16
HC = NH * NC
NN = 10000
EE = 320000
ACCW = 128
BB = 50
IBLK = 8

_SELU_SCALE = 1.0507009873554805
_SELU_ALPHA = 1.6732632423543772


def _selu(v):
    return _SELU_SCALE * jnp.where(v > 0, v, _SELU_ALPHA * (jnp.exp(v) - 1.0))


def _edge_pass(xl, xr, src2d, dst2d, att):
    n = xl.shape[0]
    rows_total = src2d.shape[0]
    rows_per_core = rows_total // 2
    npc = 624
    chunks = []
    off = 0
    while off < npc:
        cnt = min(48, npc - off)
        chunks.append((off, cnt))
        off += cnt
    tail_off = npc * 16
    tail_cnt = n - tail_off

    rows_per_tile = rows_per_core // 16
    tail_rows = rows_per_core - rows_per_tile * 16
    pairs = rows_per_tile // 2

    mesh = plsc.VectorSubcoreMesh(core_axis_name="c", subcore_axis_name="s")

    @functools.partial(
        pl.kernel,
        mesh=mesh,
        out_type=jax.ShapeDtypeStruct((2, n, ACCW), jnp.float32),
        scratch_types=[
            pltpu.VMEM((IBLK, BB), jnp.int32),
            pltpu.VMEM((IBLK, BB), jnp.int32),
            pltpu.VMEM((BB, 128), jnp.float32),
            pltpu.VMEM((BB, 128), jnp.float32),
            pltpu.VMEM((BB, 128), jnp.float32),
            pltpu.VMEM((BB, 128), jnp.float32),
            pltpu.VMEM((BB, ACCW), jnp.float32),
            pltpu.VMEM((BB, ACCW), jnp.float32),
            pltpu.VMEM((NH, NC), jnp.float32),
            pltpu.VMEM_SHARED((n, ACCW), jnp.float32),
            pltpu.SemaphoreType.DMA,
            pltpu.SemaphoreType.DMA,
            pltpu.SemaphoreType.DMA,
            pltpu.SemaphoreType.DMA,
            pltpu.SemaphoreType.DMA,
            pltpu.SemaphoreType.DMA,
        ],
        compiler_params=pltpu.CompilerParams(needs_layout_passes=False),
    )
    def body(xl_hbm, xr_hbm, src_hbm, dst_hbm, att_hbm, out_hbm,
             blk_s, blk_d, xlv0, xlv1, xrv0, xrv1, msgv0, msgv1,
             attv, accsh, sa0, sa1, sb0, sb1, sc0, sc1):
        xlv = (xlv0, xlv1)
        xrv = (xrv0, xrv1)
        msgv = (msgv0, msgv1)
        sems = (sa0, sa1, sb0, sb1)
        ssem = (sc0, sc1)
        c = lax.axis_index("c")
        s = lax.axis_index("s")
        pltpu.sync_copy(att_hbm, attv)
        att_rows = [attv[h] for h in range(NH)]
        lane = lax.iota(jnp.int32, 16)
        zero16 = jnp.zeros((16,), jnp.float32)

        def zbody(i, _):
            for k in range(ACCW // 16):
                msgv[0][i, pl.ds(16 * k, 16)] = zero16
            return 0
        lax.fori_loop(0, BB, zbody, 0)

        base = s * npc
        for coff, cnt in chunks:
            pltpu.sync_copy(msgv[0].at[pl.ds(0, cnt)],
                            accsh.at[pl.ds(base + coff, cnt)])

        @pl.when(s == 15)
        def _zero_tail():
            pltpu.sync_copy(msgv[0].at[pl.ds(0, tail_cnt)],
                            accsh.at[pl.ds(tail_off, tail_cnt)])
        plsc.subcore_barrier()

        start = c * rows_per_core + s * rows_per_tile

        def load_block(k):
            row0 = start + IBLK * k
            pltpu.sync_copy(src_hbm.at[pl.ds(row0, IBLK)], blk_s)
            pltpu.sync_copy(dst_hbm.at[pl.ds(row0, IBLK)], blk_d)

        def issue_gathers(b, j):
            jj = b % IBLK
            pltpu.async_copy(xl_hbm.at[blk_s.at[jj]], xlv[j], sems[2 * j])
            pltpu.async_copy(xr_hbm.at[blk_d.at[jj]], xrv[j], sems[2 * j + 1])

        def wait_gathers(j):
            pltpu.make_async_copy(xl_hbm.at[blk_s.at[0]], xlv[j], sems[2 * j]).wait()
            pltpu.make_async_copy(xr_hbm.at[blk_d.at[0]], xrv[j], sems[2 * j + 1]).wait()

        def scatter_sync(b, j):
            jj = b % IBLK
            pltpu.sync_copy(msgv[j], accsh.at[blk_d.at[jj]], add=True)

        def compute_batch(j):
            xlb, xrb, msgb = xlv[j], xrv[j], msgv[j]

            def edge_body(i, _):
                for u in range(5):
                    e = i * 5 + u
                    svec = jnp.zeros((16,), jnp.float32)
                    xls = []
                    for h in range(NH):
                        a = xlb[e, pl.ds(16 * h, 16)]
                        b = xrb[e, pl.ds(16 * h, 16)]
                        t = a + b
                        t = jnp.maximum(t, 0.2 * t)
                        t = t * att_rows[h]
                        sc = jnp.sum(t)
                        svec = jnp.where(lane == h, sc, svec)
                        xls.append(a)
                    ex = jnp.where(lane < NH, jnp.exp(svec), 0.0)
                    msgb[e, pl.ds(HC, 16)] = ex
                    for h in range(NH):
                        msgb[e, pl.ds(16 * h, 16)] = xls[h] * ex[h]
                return 0
            lax.fori_loop(0, BB // 5, edge_body, 0)

        bpp = IBLK // 2

        def pair_body(g, _):
            b_a = 2 * g

            @pl.when(g % bpp == 0)
            def _reload_block():
                load_block(g // bpp)
                issue_gathers(b_a, 0)

            issue_gathers(b_a + 1, 1)
            wait_gathers(0)
            compute_batch(0)
            scatter_sync(b_a, 0)

            @pl.when((g + 1 < pairs) & ((g + 1) % bpp != 0))
            def _stage_next_a():
                issue_gathers(b_a + 2, 0)

            wait_gathers(1)
            compute_batch(1)
            scatter_sync(b_a + 1, 1)
            return 0
        lax.fori_loop(0, pairs, pair_body, 0)
        plsc.subcore_barrier()

        for coff, cnt in chunks:
            pltpu.sync_copy(accsh.at[pl.ds(base + coff, cnt)],
                            out_hbm.at[c, pl.ds(base + coff, cnt)])

        @pl.when(s == 15)
        def _dump_tail():
            pltpu.sync_copy(accsh.at[pl.ds(tail_off, tail_cnt)],
                            out_hbm.at[c, pl.ds(tail_off, tail_cnt)])

    return body(xl, xr, src2d, dst2d, att)


def _lin2(x, wl, bl, wr, br):
    n, d = x.shape
    o = wl.shape[1]
    blk = 2000

    def body(x_ref, wl_ref, bl_ref, wr_ref, br_ref, ol_ref, or_ref):
        xx = x_ref[...]
        ol_ref[...] = jnp.dot(xx, wl_ref[...], preferred_element_type=jnp.float32) + bl_ref[...]
        or_ref[...] = jnp.dot(xx, wr_ref[...], preferred_element_type=jnp.float32) + br_ref[...]

    return pl.pallas_call(
        body,
        grid=(n // blk,),
        in_specs=[
            pl.BlockSpec((blk, d), lambda i: (i, 0)),
            pl.BlockSpec((d, o), lambda i: (0, 0)),
            pl.BlockSpec((o,), lambda i: (0,)),
            pl.BlockSpec((d, o), lambda i: (0, 0)),
            pl.BlockSpec((o,), lambda i: (0,)),
        ],
        out_specs=[
            pl.BlockSpec((blk, o), lambda i: (i, 0)),
            pl.BlockSpec((blk, o), lambda i: (i, 0)),
        ],
        out_shape=[jax.ShapeDtypeStruct((n, o), jnp.float32)] * 2,
    )(x, wl, bl, wr, br)


def _normalize_block(p, bias):
    cols = []
    for h in range(NH):
        dh = p[:, HC + h:HC + h + 1]
        cols.append(p[:, 16 * h:16 * h + 16] / (dh + 1e-16))
    return _selu(jnp.concatenate(cols, axis=1) + bias)


def _finalize_pre2(accout, bias, wl, bl, wr, br):
    n = accout.shape[1]
    o = wl.shape[1]
    blk = 2000

    def body(a_ref, bias_ref, wl_ref, bl_ref, wr_ref, br_ref,
             h_ref, ol_ref, or_ref):
        h1 = _normalize_block(a_ref[0] + a_ref[1], bias_ref[...])
        h_ref[...] = h1
        ol_ref[...] = jnp.dot(h1, wl_ref[...], preferred_element_type=jnp.float32) + bl_ref[...]
        or_ref[...] = jnp.dot(h1, wr_ref[...], preferred_element_type=jnp.float32) + br_ref[...]

    return pl.pallas_call(
        body,
        grid=(n // blk,),
        in_specs=[
            pl.BlockSpec((2, blk, ACCW), lambda i: (0, i, 0)),
            pl.BlockSpec((HC,), lambda i: (0,)),
            pl.BlockSpec((HC, o), lambda i: (0, 0)),
            pl.BlockSpec((o,), lambda i: (0,)),
            pl.BlockSpec((HC, o), lambda i: (0, 0)),
            pl.BlockSpec((o,), lambda i: (0,)),
        ],
        out_specs=[
            pl.BlockSpec((blk, HC), lambda i: (i, 0)),
            pl.BlockSpec((blk, o), lambda i: (i, 0)),
            pl.BlockSpec((blk, o), lambda i: (i, 0)),
        ],
        out_shape=[jax.ShapeDtypeStruct((n, HC), jnp.float32),
                   jax.ShapeDtypeStruct((n, o), jnp.float32),
                   jax.ShapeDtypeStruct((n, o), jnp.float32)],
    )(accout, bias, wl, bl, wr, br)


def _finalize_lin(accout, bias, h1, lin_w, lin_b):
    n = accout.shape[1]
    o = lin_w.shape[1]
    blk = 2000

    def body(a_ref, bias_ref, h1_ref, w_ref, b_ref, o_ref):
        h2 = _normalize_block(a_ref[0] + a_ref[1], bias_ref[...])
        h = jnp.concatenate([h1_ref[...], h2], axis=1)
        o_ref[...] = _selu(
            jnp.dot(h, w_ref[...], preferred_element_type=jnp.float32) + b_ref[...])

    return pl.pallas_call(
        body,
        grid=(n // blk,),
        in_specs=[
            pl.BlockSpec((2, blk, ACCW), lambda i: (0, i, 0)),
            pl.BlockSpec((HC,), lambda i: (0,)),
            pl.BlockSpec((blk, HC), lambda i: (i, 0)),
            pl.BlockSpec((2 * HC, o), lambda i: (0, 0)),
            pl.BlockSpec((o,), lambda i: (0,)),
        ],
        out_specs=pl.BlockSpec((blk, o), lambda i: (i, 0)),
        out_shape=jax.ShapeDtypeStruct((n, o), jnp.float32),
    )(accout, bias, h1, lin_w, lin_b)


def _mlp(t, w1, b1, w2, b2):
    m, k = t.shape
    mid = w1.shape[1]
    o = w2.shape[1]
    blk = 4000

    def body(t_ref, w1_ref, b1_ref, w2_ref, b2_ref, o_ref):
        u = _selu(jnp.dot(t_ref[...], w1_ref[...], preferred_element_type=jnp.float32) + b1_ref[...])
        o_ref[...] = jnp.dot(u, w2_ref[...], preferred_element_type=jnp.float32) + b2_ref[...]

    return pl.pallas_call(
        body,
        grid=(m // blk,),
        in_specs=[
            pl.BlockSpec((blk, k), lambda i: (i, 0)),
            pl.BlockSpec((k, mid), lambda i: (0, 0)),
            pl.BlockSpec((mid,), lambda i: (0,)),
            pl.BlockSpec((mid, o), lambda i: (0, 0)),
            pl.BlockSpec((o,), lambda i: (0,)),
        ],
        out_specs=pl.BlockSpec((blk, o), lambda i: (i, 0)),
        out_shape=jax.ShapeDtypeStruct((m, o), jnp.float32),
    )(t, w1, b1, w2, b2)


def _pad_w(w, b):
    o = w.shape[1]
    return jnp.pad(w, ((0, 0), (0, 128 - o))), jnp.pad(b, (0, 128 - o))


def kernel(x, edge_index, edge_weight, params):
    p = params
    src2d = edge_index[0].reshape(EE // BB, BB)
    dst2d = edge_index[1].reshape(EE // BB, BB)

    wl1, bl1 = _pad_w(p["g1_Wl"], p["g1_bl"])
    wr1, br1 = _pad_w(p["g1_Wr"], p["g1_br"])
    wl2, bl2 = _pad_w(p["g2_Wl"], p["g2_bl"])
    wr2, br2 = _pad_w(p["g2_Wr"], p["g2_br"])

    xl1, xr1 = _lin2(x, wl1, bl1, wr1, br1)
    acc1 = _edge_pass(xl1, xr1, src2d, dst2d, p["g1_att"])
    h1, xl2, xr2 = _finalize_pre2(acc1, p["g1_bias"], wl2, bl2, wr2, br2)
    acc2 = _edge_pass(xl2, xr2, src2d, dst2d, p["g2_att"])
    h3 = _finalize_lin(acc2, p["g2_bias"], h1, p["lin_W"], p["lin_b"])
    t = h3.reshape(-1, 38, 128).transpose(0, 2, 1).reshape(-1, 38)
    out = _mlp(t, p["l1_W"], p["l1_b"], p["l2_W"], p["l2_b"])
    return out.reshape(-1, 128, 37)


# speedup vs baseline: 1.0624x; 1.0624x over previous
import functools

import jax
import jax.numpy as jnp
from jax import lax
from jax.experimental import pallas as pl
from jax.experimental.pallas import tpu as pltpu
from jax.experimental.pallas import tpu_sc as plsc

NH = 5
NC = 16
HC = NH * NC
NN = 10000
EE = 320000
ACCW = 128
BB = 50
IBLK = 8

_SELU_SCALE = 1.0507009873554805
_SELU_ALPHA = 1.6732632423543772


def _selu(v):
    return _SELU_SCALE * jnp.where(v > 0, v, _SELU_ALPHA * (jnp.exp(v) - 1.0))


def _edge_pass(xl, xr, src2d, dst2d, att):
    n = xl.shape[0]
    rows_total = src2d.shape[0]
    rows_per_core = rows_total // 2
    npc = 624
    chunks = []
    off = 0
    while off < npc:
        cnt = min(48, npc - off)
        chunks.append((off, cnt))
        off += cnt
    tail_off = npc * 16
    tail_cnt = n - tail_off

    rows_per_tile = rows_per_core // 16
    tail_rows = rows_per_core - rows_per_tile * 16
    pairs = rows_per_tile // 2

    mesh = plsc.VectorSubcoreMesh(core_axis_name="c", subcore_axis_name="s")

    @functools.partial(
        pl.kernel,
        mesh=mesh,
        out_type=jax.ShapeDtypeStruct((2, n, ACCW), jnp.float32),
        scratch_types=[
            pltpu.VMEM((IBLK, BB), jnp.int32),
            pltpu.VMEM((IBLK, BB), jnp.int32),
            pltpu.VMEM((BB, 128), jnp.float32),
            pltpu.VMEM((BB, 128), jnp.float32),
            pltpu.VMEM((BB, 128), jnp.float32),
            pltpu.VMEM((BB, 128), jnp.float32),
            pltpu.VMEM((BB, ACCW), jnp.float32),
            pltpu.VMEM((BB, ACCW), jnp.float32),
            pltpu.VMEM((NH, NC), jnp.float32),
            pltpu.VMEM_SHARED((n, ACCW), jnp.float32),
            pltpu.SemaphoreType.DMA,
            pltpu.SemaphoreType.DMA,
            pltpu.SemaphoreType.DMA,
            pltpu.SemaphoreType.DMA,
            pltpu.SemaphoreType.DMA,
            pltpu.SemaphoreType.DMA,
        ],
        compiler_params=pltpu.CompilerParams(needs_layout_passes=False),
    )
    def body(xl_hbm, xr_hbm, src_hbm, dst_hbm, att_hbm, out_hbm,
             blk_s, blk_d, xlv0, xlv1, xrv0, xrv1, msgv0, msgv1,
             attv, accsh, sa0, sa1, sb0, sb1, sc0, sc1):
        xlv = (xlv0, xlv1)
        xrv = (xrv0, xrv1)
        msgv = (msgv0, msgv1)
        sems = (sa0, sa1, sb0, sb1)
        ssem = (sc0, sc1)
        c = lax.axis_index("c")
        s = lax.axis_index("s")
        pltpu.sync_copy(att_hbm, attv)
        att_rows = [attv[h] for h in range(NH)]
        lane = lax.iota(jnp.int32, 16)
        zero16 = jnp.zeros((16,), jnp.float32)

        def zbody(i, _):
            for k in range(ACCW // 16):
                msgv[0][i, pl.ds(16 * k, 16)] = zero16
            return 0
        lax.fori_loop(0, BB, zbody, 0)

        base = s * npc
        for coff, cnt in chunks:
            pltpu.sync_copy(msgv[0].at[pl.ds(0, cnt)],
                            accsh.at[pl.ds(base + coff, cnt)])

        @pl.when(s == 15)
        def _zero_tail():
            pltpu.sync_copy(msgv[0].at[pl.ds(0, tail_cnt)],
                            accsh.at[pl.ds(tail_off, tail_cnt)])
        plsc.subcore_barrier()

        start = c * rows_per_core + s * rows_per_tile

        def load_block(k):
            row0 = start + IBLK * k
            pltpu.sync_copy(src_hbm.at[pl.ds(row0, IBLK)], blk_s)
            pltpu.sync_copy(dst_hbm.at[pl.ds(row0, IBLK)], blk_d)

        def issue_gathers(b, j):
            jj = b % IBLK
            pltpu.async_copy(xl_hbm.at[blk_s.at[jj]], xlv[j], sems[2 * j])
            pltpu.async_copy(xr_hbm.at[blk_d.at[jj]], xrv[j], sems[2 * j + 1])

        def wait_gathers(j):
            pltpu.make_async_copy(xl_hbm.at[blk_s.at[0]], xlv[j], sems[2 * j]).wait()
            pltpu.make_async_copy(xr_hbm.at[blk_d.at[0]], xrv[j], sems[2 * j + 1]).wait()

        def issue_scatter(b, j):
            jj = b % IBLK
            pltpu.async_copy(msgv[j], accsh.at[blk_d.at[jj]], ssem[j], add=True)

        def wait_scatter(j):
            pltpu.make_async_copy(msgv[j], accsh.at[blk_d.at[0]], ssem[j]).wait()

        def compute_batch(j):
            xlb, xrb, msgb = xlv[j], xrv[j], msgv[j]

            def edge_body(i, _):
                for u in range(5):
                    e = i * 5 + u
                    svec = jnp.zeros((16,), jnp.float32)
                    xls = []
                    for h in range(NH):
                        a = xlb[e, pl.ds(16 * h, 16)]
                        b = xrb[e, pl.ds(16 * h, 16)]
                        t = a + b
                        t = jnp.maximum(t, 0.2 * t)
                        t = t * att_rows[h]
                        sc = jnp.sum(t)
                        svec = jnp.where(lane == h, sc, svec)
                        xls.append(a)
                    ex = jnp.where(lane < NH, jnp.exp(svec), 0.0)
                    msgb[e, pl.ds(HC, 16)] = ex
                    for h in range(NH):
                        msgb[e, pl.ds(16 * h, 16)] = xls[h] * ex[h]
                return 0
            lax.fori_loop(0, BB // 5, edge_body, 0)

        bpp = IBLK // 2

        def pair_body(g, _):
            b_a = 2 * g

            @pl.when(g % bpp == 0)
            def _reload_block():
                @pl.when(g > 0)
                def _drain():
                    wait_scatter(1)
                load_block(g // bpp)
                issue_gathers(b_a, 0)

            issue_gathers(b_a + 1, 1)
            wait_gathers(0)
            compute_batch(0)

            @pl.when((g > 0) & (g % bpp != 0))
            def _wait_prev_b():
                wait_scatter(1)
            issue_scatter(b_a, 0)

            @pl.when((g + 1 < pairs) & ((g + 1) % bpp != 0))
            def _stage_next_a():
                issue_gathers(b_a + 2, 0)

            wait_gathers(1)
            compute_batch(1)
            wait_scatter(0)
            issue_scatter(b_a + 1, 1)
            return 0
        lax.fori_loop(0, pairs, pair_body, 0)
        wait_scatter(1)
        plsc.subcore_barrier()

        for coff, cnt in chunks:
            pltpu.sync_copy(accsh.at[pl.ds(base + coff, cnt)],
                            out_hbm.at[c, pl.ds(base + coff, cnt)])

        @pl.when(s == 15)
        def _dump_tail():
            pltpu.sync_copy(accsh.at[pl.ds(tail_off, tail_cnt)],
                            out_hbm.at[c, pl.ds(tail_off, tail_cnt)])

    return body(xl, xr, src2d, dst2d, att)


def _lin2(x, wl, bl, wr, br):
    n, d = x.shape
    o = wl.shape[1]
    blk = 2000

    def body(x_ref, wl_ref, bl_ref, wr_ref, br_ref, ol_ref, or_ref):
        xx = x_ref[...]
        ol_ref[...] = jnp.dot(xx, wl_ref[...], preferred_element_type=jnp.float32) + bl_ref[...]
        or_ref[...] = jnp.dot(xx, wr_ref[...], preferred_element_type=jnp.float32) + br_ref[...]

    return pl.pallas_call(
        body,
        grid=(n // blk,),
        in_specs=[
            pl.BlockSpec((blk, d), lambda i: (i, 0)),
            pl.BlockSpec((d, o), lambda i: (0, 0)),
            pl.BlockSpec((o,), lambda i: (0,)),
            pl.BlockSpec((d, o), lambda i: (0, 0)),
            pl.BlockSpec((o,), lambda i: (0,)),
        ],
        out_specs=[
            pl.BlockSpec((blk, o), lambda i: (i, 0)),
            pl.BlockSpec((blk, o), lambda i: (i, 0)),
        ],
        out_shape=[jax.ShapeDtypeStruct((n, o), jnp.float32)] * 2,
    )(x, wl, bl, wr, br)


def _normalize_block(p, bias):
    cols = []
    for h in range(NH):
        dh = p[:, HC + h:HC + h + 1]
        cols.append(p[:, 16 * h:16 * h + 16] / (dh + 1e-16))
    return _selu(jnp.concatenate(cols, axis=1) + bias)


def _finalize_pre2(accout, bias, wl, bl, wr, br):
    n = accout.shape[1]
    o = wl.shape[1]
    blk = 2000

    def body(a_ref, bias_ref, wl_ref, bl_ref, wr_ref, br_ref,
             h_ref, ol_ref, or_ref):
        h1 = _normalize_block(a_ref[0] + a_ref[1], bias_ref[...])
        h_ref[...] = h1
        ol_ref[...] = jnp.dot(h1, wl_ref[...], preferred_element_type=jnp.float32) + bl_ref[...]
        or_ref[...] = jnp.dot(h1, wr_ref[...], preferred_element_type=jnp.float32) + br_ref[...]

    return pl.pallas_call(
        body,
        grid=(n // blk,),
        in_specs=[
            pl.BlockSpec((2, blk, ACCW), lambda i: (0, i, 0)),
            pl.BlockSpec((HC,), lambda i: (0,)),
            pl.BlockSpec((HC, o), lambda i: (0, 0)),
            pl.BlockSpec((o,), lambda i: (0,)),
            pl.BlockSpec((HC, o), lambda i: (0, 0)),
            pl.BlockSpec((o,), lambda i: (0,)),
        ],
        out_specs=[
            pl.BlockSpec((blk, HC), lambda i: (i, 0)),
            pl.BlockSpec((blk, o), lambda i: (i, 0)),
            pl.BlockSpec((blk, o), lambda i: (i, 0)),
        ],
        out_shape=[jax.ShapeDtypeStruct((n, HC), jnp.float32),
                   jax.ShapeDtypeStruct((n, o), jnp.float32),
                   jax.ShapeDtypeStruct((n, o), jnp.float32)],
    )(accout, bias, wl, bl, wr, br)


def _finalize_lin(accout, bias, h1, lin_w, lin_b):
    n = accout.shape[1]
    o = lin_w.shape[1]
    blk = 2000

    def body(a_ref, bias_ref, h1_ref, w_ref, b_ref, o_ref):
        h2 = _normalize_block(a_ref[0] + a_ref[1], bias_ref[...])
        h = jnp.concatenate([h1_ref[...], h2], axis=1)
        o_ref[...] = _selu(
            jnp.dot(h, w_ref[...], preferred_element_type=jnp.float32) + b_ref[...])

    return pl.pallas_call(
        body,
        grid=(n // blk,),
        in_specs=[
            pl.BlockSpec((2, blk, ACCW), lambda i: (0, i, 0)),
            pl.BlockSpec((HC,), lambda i: (0,)),
            pl.BlockSpec((blk, HC), lambda i: (i, 0)),
            pl.BlockSpec((2 * HC, o), lambda i: (0, 0)),
            pl.BlockSpec((o,), lambda i: (0,)),
        ],
        out_specs=pl.BlockSpec((blk, o), lambda i: (i, 0)),
        out_shape=jax.ShapeDtypeStruct((n, o), jnp.float32),
    )(accout, bias, h1, lin_w, lin_b)


def _mlp(t, w1, b1, w2, b2):
    m, k = t.shape
    mid = w1.shape[1]
    o = w2.shape[1]
    blk = 4000

    def body(t_ref, w1_ref, b1_ref, w2_ref, b2_ref, o_ref):
        u = _selu(jnp.dot(t_ref[...], w1_ref[...], preferred_element_type=jnp.float32) + b1_ref[...])
        o_ref[...] = jnp.dot(u, w2_ref[...], preferred_element_type=jnp.float32) + b2_ref[...]

    return pl.pallas_call(
        body,
        grid=(m // blk,),
        in_specs=[
            pl.BlockSpec((blk, k), lambda i: (i, 0)),
            pl.BlockSpec((k, mid), lambda i: (0, 0)),
            pl.BlockSpec((mid,), lambda i: (0,)),
            pl.BlockSpec((mid, o), lambda i: (0, 0)),
            pl.BlockSpec((o,), lambda i: (0,)),
        ],
        out_specs=pl.BlockSpec((blk, o), lambda i: (i, 0)),
        out_shape=jax.ShapeDtypeStruct((m, o), jnp.float32),
    )(t, w1, b1, w2, b2)


def _pad_w(w, b):
    o = w.shape[1]
    return jnp.pad(w, ((0, 0), (0, 128 - o))), jnp.pad(b, (0, 128 - o))


def kernel(x, edge_index, edge_weight, params):
    p = params
    src2d = edge_index[0].reshape(EE // BB, BB)
    dst2d = edge_index[1].reshape(EE // BB, BB)

    wl1, bl1 = _pad_w(p["g1_Wl"], p["g1_bl"])
    wr1, br1 = _pad_w(p["g1_Wr"], p["g1_br"])
    wl2, bl2 = _pad_w(p["g2_Wl"], p["g2_bl"])
    wr2, br2 = _pad_w(p["g2_Wr"], p["g2_br"])

    xl1, xr1 = _lin2(x, wl1, bl1, wr1, br1)
    acc1 = _edge_pass(xl1, xr1, src2d, dst2d, p["g1_att"])
    h1, xl2, xr2 = _finalize_pre2(acc1, p["g1_bias"], wl2, bl2, wr2, br2)
    acc2 = _edge_pass(xl2, xr2, src2d, dst2d, p["g2_att"])
    h3 = _finalize_lin(acc2, p["g2_bias"], h1, p["lin_W"], p["lin_b"])
    t = h3.reshape(-1, 38, 128).transpose(0, 2, 1).reshape(-1, 38)
    out = _mlp(t, p["l1_W"], p["l1_b"], p["l2_W"], p["l2_b"])
    return out.reshape(-1, 128, 37)
